# baseline (speedup 1.0000x reference)
import functools

import jax
import jax.numpy as jnp
from jax.experimental import pallas as pl
from jax.experimental.pallas import tpu as pltpu


def _round_up(x, m):
    return (x + m - 1) // m * m


def _support_bf16_kernel(x_ref, w_ref, s_ref):
    s_ref[...] = jnp.dot(
        x_ref[...].astype(jnp.bfloat16), w_ref[...].astype(jnp.bfloat16),
        preferred_element_type=jnp.float32
    ).astype(jnp.bfloat16)


def _make_support(x, weight, n_p, f_in_p, f_out_p):
    tm1 = max(d for d in (2048, 1024, 512, 256, 128) if n_p % d == 0)
    ws1 = 2 * (tm1 * f_in_p + f_in_p * f_out_p) * 4 + 2 * tm1 * f_out_p * 2
    return pl.pallas_call(
        _support_bf16_kernel,
        out_shape=jax.ShapeDtypeStruct((n_p, f_out_p), jnp.bfloat16),
        grid=(n_p // tm1,),
        in_specs=[
            pl.BlockSpec((tm1, f_in_p), lambda i: (i, 0)),
            pl.BlockSpec((f_in_p, f_out_p), lambda i: (0, 0)),
        ],
        out_specs=pl.BlockSpec((tm1, f_out_p), lambda i: (i, 0)),
        compiler_params=pltpu.CompilerParams(
            dimension_semantics=("parallel",),
            vmem_limit_bytes=int(min(max(ws1 * 2, 16 << 20), 48 << 20))),
        cost_estimate=pl.CostEstimate(
            flops=2 * n_p * f_in_p * f_out_p,
            transcendentals=0,
            bytes_accessed=int(n_p * f_in_p * 4 + f_in_p * f_out_p * 4
                               + n_p * f_out_p * 2)),
    )(x, weight)


def _stage2_outer_kernel(s_ref, b_ref, adj_ref, o_ref, *,
                         tm, n_p, f_out_p, n_stripes):
    core = pl.program_id(0)
    rows = n_stripes * tm

    def body(a_ref, o_blk):
        a = a_ref[...].astype(jnp.bfloat16)
        acc = jnp.dot(a, s_ref[...], preferred_element_type=jnp.float32)
        o_blk[...] = acc + b_ref[...]

    pltpu.emit_pipeline(
        body,
        grid=(n_stripes,),
        in_specs=[pl.BlockSpec((tm, n_p), lambda j: (j, 0))],
        out_specs=[pl.BlockSpec((tm, f_out_p), lambda j: (j, 0))],
    )(adj_ref.at[pl.ds(core * rows, rows), :],
      o_ref.at[pl.ds(core * rows, rows), :])


def kernel(x, weight, adj, bias=None):
    n, f_in = x.shape
    f_out = weight.shape[1]
    f32 = jnp.float32

    f_out_p = _round_up(f_out, 128)
    f_in_p = _round_up(f_in, 128)
    n_p = _round_up(n, 128)

    x_p = x.astype(f32)
    if (n, f_in) != (n_p, f_in_p):
        x_p = jnp.zeros((n_p, f_in_p), f32).at[:n, :f_in].set(x_p)
    w_p = weight.astype(f32)
    if (f_in, f_out) != (f_in_p, f_out_p):
        w_p = jnp.zeros((f_in_p, f_out_p), f32).at[:f_in, :f_out].set(w_p)
    adj_p = adj
    if n != n_p:
        adj_p = jnp.zeros((n_p, n_p), adj.dtype).at[:n, :n].set(adj)
    if bias is None:
        b_p = jnp.zeros((1, f_out_p), f32)
    else:
        b_p = bias.reshape(1, f_out).astype(f32)
        if f_out != f_out_p:
            b_p = jnp.zeros((1, f_out_p), f32).at[:, :f_out].set(b_p)

    support = _make_support(x_p, w_p, n_p, f_in_p, f_out_p)

    tm = next(d for d in (512, 256, 128) if n_p % d == 0)
    n_tiles = n_p // tm
    n_par = 2 if n_tiles % 2 == 0 else 1
    n_stripes = n_tiles // n_par

    ws2 = (2 * tm * n_p * 4 + n_p * f_out_p * 2
           + 2 * tm * f_out_p * 4 + f_out_p * 4)

    kfn = functools.partial(_stage2_outer_kernel, tm=tm, n_p=n_p,
                            f_out_p=f_out_p, n_stripes=n_stripes)
    out = pl.pallas_call(
        kfn,
        out_shape=jax.ShapeDtypeStruct((n_p, f_out_p), f32),
        grid=(n_par,),
        in_specs=[
            pl.BlockSpec((n_p, f_out_p), lambda i: (0, 0)),
            pl.BlockSpec((1, f_out_p), lambda i: (0, 0)),
            pl.BlockSpec(memory_space=pl.ANY),
        ],
        out_specs=pl.BlockSpec(memory_space=pl.ANY),
        compiler_params=pltpu.CompilerParams(
            dimension_semantics=("parallel",),
            vmem_limit_bytes=int(min(max(int(ws2 * 1.25), 16 << 20), 56 << 20))),
        cost_estimate=pl.CostEstimate(
            flops=2 * n_p * n_p * f_out_p,
            transcendentals=0,
            bytes_accessed=int(n_p * n_p * 4
                               + n_p * f_out_p * 2 + n_p * f_out_p * 4)),
    )(support, b_p, adj_p)

    if (n, f_out) != (n_p, f_out_p):
        out = out[:n, :f_out]
    return out


# speedup vs baseline: 1.1056x; 1.1056x over previous
import jax
import jax.numpy as jnp
from jax.experimental import pallas as pl
from jax.experimental.pallas import tpu as pltpu


def _round_up(x, m):
    return (x + m - 1) // m * m


def _support_bf16_kernel(x_ref, w_ref, s_ref):
    s_ref[...] = jnp.dot(
        x_ref[...].astype(jnp.bfloat16), w_ref[...].astype(jnp.bfloat16),
        preferred_element_type=jnp.float32
    ).astype(jnp.bfloat16)


def _adj_matmul_kernel(adj_ref, s_ref, b_ref, o_ref):
    a = adj_ref[...].astype(jnp.bfloat16)
    acc = jnp.dot(a, s_ref[...], preferred_element_type=jnp.float32)
    o_ref[...] = acc + b_ref[...]


def _adj_matmul_kernel_nobias(adj_ref, s_ref, o_ref):
    a = adj_ref[...].astype(jnp.bfloat16)
    o_ref[...] = jnp.dot(a, s_ref[...], preferred_element_type=jnp.float32)


def kernel(x, weight, adj, bias=None):
    n, f_in = x.shape
    f_out = weight.shape[1]
    f32 = jnp.float32

    f_out_p = _round_up(f_out, 128)
    f_in_p = _round_up(f_in, 128)
    n_p = _round_up(n, 128)

    x_p = x.astype(f32)
    if (n, f_in) != (n_p, f_in_p):
        x_p = jnp.zeros((n_p, f_in_p), f32).at[:n, :f_in].set(x_p)
    w_p = weight.astype(f32)
    if (f_in, f_out) != (f_in_p, f_out_p):
        w_p = jnp.zeros((f_in_p, f_out_p), f32).at[:f_in, :f_out].set(w_p)
    adj_p = adj
    if n != n_p:
        adj_p = jnp.zeros((n_p, n_p), adj.dtype).at[:n, :n].set(adj)
    has_bias = bias is not None
    if has_bias:
        b_p = bias.reshape(1, f_out).astype(f32)
        if f_out != f_out_p:
            b_p = jnp.zeros((1, f_out_p), f32).at[:, :f_out].set(b_p)

    tm1 = max(d for d in (2048, 1024, 512, 256, 128) if n_p % d == 0)
    ws1 = 2 * (tm1 * f_in_p + f_in_p * f_out_p) * 4 + 2 * tm1 * f_out_p * 2
    support = pl.pallas_call(
        _support_bf16_kernel,
        out_shape=jax.ShapeDtypeStruct((n_p, f_out_p), jnp.bfloat16),
        grid=(n_p // tm1,),
        in_specs=[
            pl.BlockSpec((tm1, f_in_p), lambda i: (i, 0)),
            pl.BlockSpec((f_in_p, f_out_p), lambda i: (0, 0)),
        ],
        out_specs=pl.BlockSpec((tm1, f_out_p), lambda i: (i, 0)),
        compiler_params=pltpu.CompilerParams(
            dimension_semantics=("parallel",),
            vmem_limit_bytes=int(min(max(ws1 * 2, 16 << 20), 48 << 20))),
        cost_estimate=pl.CostEstimate(
            flops=2 * n_p * f_in_p * f_out_p,
            transcendentals=0,
            bytes_accessed=int(n_p * f_in_p * 4 + f_in_p * f_out_p * 4
                               + n_p * f_out_p * 2)),
    )(x_p, w_p)

    tm = max(d for d in (512, 256, 128) if n_p % d == 0)
    ws2 = (2 * tm * n_p * adj_p.dtype.itemsize
           + n_p * f_out_p * 2
           + 2 * tm * f_out_p * 4
           + f_out_p * 4)
    if has_bias:
        kfn = _adj_matmul_kernel
        in_specs = [
            pl.BlockSpec((tm, n_p), lambda i: (i, 0)),
            pl.BlockSpec((n_p, f_out_p), lambda i: (0, 0)),
            pl.BlockSpec((1, f_out_p), lambda i: (0, 0)),
        ]
        args = (adj_p, support, b_p)
    else:
        kfn = _adj_matmul_kernel_nobias
        in_specs = [
            pl.BlockSpec((tm, n_p), lambda i: (i, 0)),
            pl.BlockSpec((n_p, f_out_p), lambda i: (0, 0)),
        ]
        args = (adj_p, support)

    out = pl.pallas_call(
        kfn,
        out_shape=jax.ShapeDtypeStruct((n_p, f_out_p), f32),
        grid=(n_p // tm,),
        in_specs=in_specs,
        out_specs=pl.BlockSpec((tm, f_out_p), lambda i: (i, 0)),
        compiler_params=pltpu.CompilerParams(
            dimension_semantics=("parallel",),
            vmem_limit_bytes=int(min(max(int(ws2 * 1.25), 16 << 20), 56 << 20))),
        cost_estimate=pl.CostEstimate(
            flops=2 * n_p * n_p * f_out_p,
            transcendentals=0,
            bytes_accessed=int(n_p * n_p * adj_p.dtype.itemsize
                               + n_p * f_out_p * 2 + n_p * f_out_p * 4)),
    )(*args)

    if (n, f_out) != (n_p, f_out_p):
        out = out[:n, :f_out]
    return out
